# Initial kernel scaffold; baseline (speedup 1.0000x reference)
import jax
import jax.numpy as jnp
from jax.experimental import pallas as pl


def kernel(g, lg, x, y, z, W1, b1, bn1_g, bn1_b, W2, b2, bn2_g, bn2_b):
    raise NotImplementedError("write your pallas kernel here")


# speedup vs baseline: 1.5995x; 1.5995x over previous
import functools

import jax
import jax.numpy as jnp
from jax.experimental import pallas as pl
from jax.experimental.pallas import tpu as pltpu

N = 10000
E = 160000
E_LG = 320000
D = 128


def _mm_tables_body(x_ref, w_ref, b_ref, a_ref, bu_ref, c_ref):
    t = jnp.dot(x_ref[...], w_ref[...], preferred_element_type=jnp.float32)
    t = t + b_ref[...]
    a_ref[...] = t[:, :D]
    bu_ref[...] = t[:, D:3 * D]
    c_ref[...] = t[:, 3 * D:]


def _mm_tables(x, Wcat, bcat, blk):
    V = x.shape[0]
    assert V % blk == 0
    return pl.pallas_call(
        _mm_tables_body,
        grid=(V // blk,),
        in_specs=[
            pl.BlockSpec((blk, D), lambda r: (r, 0)),
            pl.BlockSpec((D, 4 * D), lambda r: (0, 0)),
            pl.BlockSpec((1, 4 * D), lambda r: (0, 0)),
        ],
        out_specs=[
            pl.BlockSpec((blk, D), lambda r: (r, 0)),
            pl.BlockSpec((blk, 2 * D), lambda r: (r, 0)),
            pl.BlockSpec((blk, D), lambda r: (r, 0)),
        ],
        out_shape=[
            jax.ShapeDtypeStruct((V, D), jnp.float32),
            jax.ShapeDtypeStruct((V, 2 * D), jnp.float32),
            jax.ShapeDtypeStruct((V, D), jnp.float32),
        ],
    )(x, Wcat, bcat.reshape(1, 4 * D))


def _mm_bias_body(x_ref, w_ref, b_ref, o_ref):
    o_ref[...] = jnp.dot(x_ref[...], w_ref[...],
                         preferred_element_type=jnp.float32) + b_ref[...]


def _mm_bias(x, W, b, blk):
    V = x.shape[0]
    assert V % blk == 0
    return pl.pallas_call(
        _mm_bias_body,
        grid=(V // blk,),
        in_specs=[
            pl.BlockSpec((blk, D), lambda r: (r, 0)),
            pl.BlockSpec((D, D), lambda r: (0, 0)),
            pl.BlockSpec((1, D), lambda r: (0, 0)),
        ],
        out_specs=pl.BlockSpec((blk, D), lambda r: (r, 0)),
        out_shape=jax.ShapeDtypeStruct((V, D), jnp.float32),
    )(x, W, b.reshape(1, D))


def _col_stats_body(t_ref, o_ref):
    blk = t_ref[...]
    s = jnp.sum(blk, axis=0, keepdims=True)
    q = jnp.sum(blk * blk, axis=0, keepdims=True)
    upd = jnp.concatenate([s, q, jnp.zeros((6, D), jnp.float32)], axis=0)

    @pl.when(pl.program_id(0) == 0)
    def _():
        o_ref[...] = jnp.zeros_like(o_ref)

    o_ref[...] += upd


def _col_stats(t, blk):
    V = t.shape[0]
    assert V % blk == 0
    return pl.pallas_call(
        _col_stats_body,
        grid=(V // blk,),
        in_specs=[pl.BlockSpec((blk, D), lambda r: (r, 0))],
        out_specs=pl.BlockSpec((8, D), lambda r: (0, 0)),
        out_shape=jax.ShapeDtypeStruct((8, D), jnp.float32),
    )(t)


def _bn_apply_body(count, base_ref, t_ref, st_ref, g_ref, b_ref, o_ref):
    s = st_ref[0, :]
    q = st_ref[1, :]
    mean = s / count
    var = q / count - mean * mean
    rstd = jax.lax.rsqrt(var + 1e-5)
    h = (t_ref[...] - mean) * (rstd * g_ref[...]) + b_ref[...]
    o_ref[...] = base_ref[...] + h / (1.0 + jnp.exp(-h))


def _bn_apply_residual(base, t, stats, gamma, beta, blk):
    V = t.shape[0]
    assert V % blk == 0
    t2 = t.reshape(V, D) if t.ndim == 2 else t
    return pl.pallas_call(
        functools.partial(_bn_apply_body, float(V)),
        grid=(V // blk,),
        in_specs=[
            pl.BlockSpec((blk, D), lambda r: (r, 0)),
            pl.BlockSpec((blk, D), lambda r: (r, 0)),
            pl.BlockSpec((8, D), lambda r: (0, 0)),
            pl.BlockSpec((1, D), lambda r: (0, 0)),
            pl.BlockSpec((1, D), lambda r: (0, 0)),
        ],
        out_specs=pl.BlockSpec((blk, D), lambda r: (r, 0)),
        out_shape=jax.ShapeDtypeStruct((V, D), jnp.float32),
    )(base, t2, stats, gamma.reshape(1, D), beta.reshape(1, D))


def _egc(edge_index, x, edge_attr, W, b, bn_g, bn_b, num_nodes,
         node_blk, edge_blk):
    i = edge_index[0]
    j = edge_index[1]
    Wcat = jnp.concatenate([W[0], W[1], W[4], W[3]], axis=1)
    bcat = jnp.concatenate([b[0], b[1], b[4], b[3]], axis=0)
    A, BU, C = _mm_tables(x, Wcat, bcat, node_blk)
    EY = _mm_bias(edge_attr, W[2], b[2], edge_blk)

    buj = jnp.take(BU, j, axis=0)
    em = jnp.take(A, i, axis=0) + buj[:, :D] + EY
    se = jax.nn.sigmoid(em)
    nm = se * buj[:, D:]
    SS = jax.ops.segment_sum(se, i, num_segments=num_nodes)
    SN = jax.ops.segment_sum(nm, i, num_segments=num_nodes)
    v = C + SN / (SS + 1e-9)

    em_stats = _col_stats(em, edge_blk)
    v_stats = _col_stats(v, node_blk)
    x_new = _bn_apply_residual(x, v, v_stats, bn_g[1], bn_b[1], node_blk)
    e_new = _bn_apply_residual(edge_attr, em, em_stats, bn_g[0], bn_b[0],
                               edge_blk)
    return x_new, e_new


def kernel(g, lg, x, y, z, W1, b1, bn1_g, bn1_b, W2, b2, bn2_g, bn2_b):
    x_new, m = _egc(g, x, y, W1, b1, bn1_g, bn1_b, N,
                    node_blk=400, edge_blk=1600)
    y_new, z_new = _egc(lg, m, z, W2, b2, bn2_g, bn2_b, E,
                        node_blk=1600, edge_blk=1600)
    return (x_new, y_new, z_new)
